# baseline (speedup 1.0000x reference)
import dataclasses
import functools

import jax
import jax.numpy as jnp
from jax import lax
from jax.experimental import pallas as pl
from jax.experimental.pallas import tpu as pltpu
from jax.experimental.pallas import tpu_sc as plsc

N_NODES = 10000
N_PAD = 10240
N_EDGES = 320000
D = 128
NC = 2
NS = 16
NW = NC * NS
EPW = N_EDGES // NW
CHUNK = 125
NCHUNK = EPW // CHUNK
RPT = N_PAD // NS
LANES = 16

_mesh = plsc.VectorSubcoreMesh(core_axis_name="c", subcore_axis_name="s")

_sc_params = pltpu.CompilerParams(needs_layout_passes=False,
                                  use_tc_tiling_on_sc=False)


def _counts_body(src_hbm, dst_hbm, out_hbm, src_v, dst_v, cd_v, cs_v, cl_v, sem):
    cid = lax.axis_index("c")
    sid = lax.axis_index("s")
    wid = cid * NS + sid

    zeros16 = jnp.zeros((LANES,), jnp.float32)

    @pl.loop(0, N_PAD // LANES)
    def _zero(i):
        cd_v[0, pl.ds(i * LANES, LANES)] = zeros16
        cs_v[0, pl.ds(i * LANES, LANES)] = zeros16
        cl_v[0, pl.ds(i * LANES, LANES)] = zeros16

    cp1 = pltpu.async_copy(src_hbm.at[pl.ds(wid * EPW, EPW)], src_v, sem)
    cp2 = pltpu.async_copy(dst_hbm.at[pl.ds(wid * EPW, EPW)], dst_v, sem)
    cp1.wait()
    cp2.wait()

    ones16 = jnp.ones((LANES,), jnp.float32)
    zeros16i = jnp.zeros((LANES,), jnp.int32)

    @pl.loop(0, EPW // LANES)
    def _count(i):
        s16 = src_v[pl.ds(i * LANES, LANES)]
        d16 = dst_v[pl.ds(i * LANES, LANES)]
        neq = s16 != d16
        plsc.addupdate_scatter(cd_v, [zeros16i, d16], ones16, mask=neq)
        plsc.addupdate_scatter(cs_v, [zeros16i, s16], ones16, mask=neq)
        plsc.addupdate_scatter(cl_v, [zeros16i, d16], ones16,
                               mask=jnp.logical_not(neq))

    pltpu.sync_copy(cd_v, out_hbm.at[0, pl.ds(wid, 1)])
    pltpu.sync_copy(cs_v, out_hbm.at[1, pl.ds(wid, 1)])
    pltpu.sync_copy(cl_v, out_hbm.at[2, pl.ds(wid, 1)])


@jax.jit
def _sc_counts(src_flat, dst_flat):
    return pl.kernel(
        _counts_body,
        out_type=jax.ShapeDtypeStruct((3, NW, N_PAD), jnp.float32),
        mesh=_mesh,
        scratch_types=[
            pltpu.VMEM((EPW,), jnp.int32),
            pltpu.VMEM((EPW,), jnp.int32),
            pltpu.VMEM((1, N_PAD), jnp.float32),
            pltpu.VMEM((1, N_PAD), jnp.float32),
            pltpu.VMEM((1, N_PAD), jnp.float32),
            pltpu.SemaphoreType.DMA,
        ],
        compiler_params=_sc_params,
    )(src_flat, dst_flat)


NCHUNK = EPW // CHUNK

NBUF = 8
DEPTH = NBUF // 2


def _agg_body(y_hbm, src_hbm, dst_hbm, zero_hbm, out_hbm,
              src_v, dst_v, *rest):
    bufs = rest[:NBUF]
    acc_sh, semg, sems, semz = rest[NBUF:]
    cid = lax.axis_index("c")
    sid = lax.axis_index("s")
    wid = cid * NS + sid

    zcp = pltpu.async_copy(zero_hbm, acc_sh.at[pl.ds(sid * RPT, RPT)], semz)
    cp1 = pltpu.async_copy(src_hbm.at[wid], src_v, sem=semg)
    cp2 = pltpu.async_copy(dst_hbm.at[wid], dst_v, sem=semg)
    cp1.wait()
    cp2.wait()
    zcp.wait()
    plsc.subcore_barrier()

    def gather(j, b):
        pltpu.async_copy(y_hbm.at[src_v.at[j]], bufs[b], semg)

    def drain_gather(j, b):
        pltpu.make_async_copy(y_hbm.at[src_v.at[j]], bufs[b], semg).wait()

    def scatter(j, b):
        pltpu.async_copy(bufs[b], acc_sh.at[dst_v.at[j]], sems, add=True)

    def drain_scatter(j, b):
        pltpu.make_async_copy(bufs[b], acc_sh.at[dst_v.at[j]], sems).wait()

    for b in range(DEPTH):
        gather(b, b)

    @pl.loop(0, NCHUNK, step=NBUF)
    def _edges(base):
        for b in range(NBUF):
            j = base + b
            bn = (b + DEPTH) % NBUF

            @pl.when(j >= DEPTH)
            def _():
                drain_scatter(j - DEPTH, bn)

            @pl.when(j + DEPTH < NCHUNK)
            def _():
                gather(j + DEPTH, bn)

            drain_gather(j, b)
            scatter(j, b)

    for k in range(DEPTH):
        j = NCHUNK - DEPTH + k
        drain_scatter(j, j % NBUF)

    plsc.subcore_barrier()
    pltpu.sync_copy(acc_sh.at[pl.ds(sid * RPT, RPT)],
                    out_hbm.at[cid].at[pl.ds(sid * RPT, RPT)])


@jax.jit
def _sc_agg(y_bf16, src_r, dst_r, zero_rows):
    return pl.kernel(
        _agg_body,
        out_type=jax.ShapeDtypeStruct((NC, N_PAD, D), jnp.bfloat16),
        mesh=_mesh,
        scratch_types=[
            pltpu.VMEM((NCHUNK, CHUNK), jnp.int32),
            pltpu.VMEM((NCHUNK, CHUNK), jnp.int32),
        ] + [pltpu.VMEM((CHUNK, D), jnp.bfloat16)] * NBUF + [
            pltpu.VMEM_SHARED((N_PAD, D), jnp.bfloat16),
            pltpu.SemaphoreType.DMA,
            pltpu.SemaphoreType.DMA,
            pltpu.SemaphoreType.DMA,
        ],
        compiler_params=_sc_params,
    )(y_bf16, src_r, dst_r, zero_rows)


BP = 1024


def _prep_body(cnt_ref, x_ref, y_ref, a_ref, b_ref, g_ref):
    cnt = jnp.sum(cnt_ref[...], axis=1)
    deg = jnp.maximum(cnt[0], 1.0)
    deg_s = jnp.maximum(cnt[1], 1.0)
    cself = cnt[2]
    dinv = lax.rsqrt(deg)
    dinv_s = lax.rsqrt(deg_s)
    a_ref[...] = dinv[:, None]
    b_ref[...] = dinv_s[:, None]
    g_ref[...] = (cself * dinv * dinv_s)[:, None]
    y_ref[...] = x_ref[...] * dinv_s[:, None]


@jax.jit
def _tc_prep(cnt_parts, x_pad):
    grid = (N_PAD // BP,)
    return pl.pallas_call(
        _prep_body,
        grid=grid,
        in_specs=[
            pl.BlockSpec((3, NW, BP), lambda i: (0, 0, i)),
            pl.BlockSpec((BP, D), lambda i: (i, 0)),
        ],
        out_specs=[
            pl.BlockSpec((BP, D), lambda i: (i, 0)),
            pl.BlockSpec((BP, 1), lambda i: (i, 0)),
            pl.BlockSpec((BP, 1), lambda i: (i, 0)),
            pl.BlockSpec((BP, 1), lambda i: (i, 0)),
        ],
        out_shape=[
            jax.ShapeDtypeStruct((N_PAD, D), jnp.float32),
            jax.ShapeDtypeStruct((N_PAD, 1), jnp.float32),
            jax.ShapeDtypeStruct((N_PAD, 1), jnp.float32),
            jax.ShapeDtypeStruct((N_PAD, 1), jnp.float32),
        ],
    )(cnt_parts, x_pad)


def _layer_body(h_ref, s_ref, a_ref, b_ref, g_ref, w1_ref, w2_ref,
                bias_ref, o_ref, y_ref, *, last):
    h = h_ref[...]
    ssum = s_ref[0] + s_ref[1]
    agg = a_ref[...] * ssum - g_ref[...] * h
    u = h + agg
    v = h * agg
    o = (jnp.dot(u, w1_ref[...], preferred_element_type=jnp.float32)
         + jnp.dot(v, w2_ref[...], preferred_element_type=jnp.float32)
         + bias_ref[...])
    if not last:
        o = jnp.where(o > 0, o, 0.01 * o)
        y_ref[...] = o * b_ref[...]
    o_ref[...] = o


@functools.partial(jax.jit, static_argnames=("last",))
def _tc_layer(h, s, alpha, beta, gamma, w1, w2, bias, *, last):
    grid = (N_PAD // BP,)
    out_shape = [jax.ShapeDtypeStruct((N_PAD, D), jnp.float32)]
    out_specs = [pl.BlockSpec((BP, D), lambda i: (i, 0))]
    if not last:
        out_shape.append(jax.ShapeDtypeStruct((N_PAD, D), jnp.float32))
        out_specs.append(pl.BlockSpec((BP, D), lambda i: (i, 0)))
    else:
        out_shape.append(jax.ShapeDtypeStruct((8, 128), jnp.float32))
        out_specs.append(pl.BlockSpec((8, 128), lambda i: (0, 0)))
    return pl.pallas_call(
        functools.partial(_layer_body, last=last),
        grid=grid,
        in_specs=[
            pl.BlockSpec((BP, D), lambda i: (i, 0)),
            pl.BlockSpec((NC, BP, D), lambda i: (0, i, 0)),
            pl.BlockSpec((BP, 1), lambda i: (i, 0)),
            pl.BlockSpec((BP, 1), lambda i: (i, 0)),
            pl.BlockSpec((BP, 1), lambda i: (i, 0)),
            pl.BlockSpec((D, D), lambda i: (0, 0)),
            pl.BlockSpec((D, D), lambda i: (0, 0)),
            pl.BlockSpec((1, D), lambda i: (0, 0)),
        ],
        out_specs=out_specs,
        out_shape=out_shape,
    )(h, s, alpha, beta, gamma, w1, w2, bias)


def kernel(inputs, edge_index, W1a, b1a, W2a, b2a, W1b, b1b, W2b, b2b):
    src = edge_index[0].astype(jnp.int32)
    dst = edge_index[1].astype(jnp.int32)
    src_r = src.reshape(NW, NCHUNK, CHUNK)
    dst_r = dst.reshape(NW, NCHUNK, CHUNK)
    zero_rows = jnp.zeros((RPT, D), jnp.bfloat16)
    x_pad = jnp.pad(inputs, ((0, N_PAD - N_NODES), (0, 0)))

    cnt_parts = _sc_counts(src, dst)
    y1, alpha, beta, gamma = _tc_prep(cnt_parts, x_pad)

    bias_a = (b1a + b2a).reshape(1, D)
    bias_b = (b1b + b2b).reshape(1, D)

    f32 = jnp.float32
    bf16 = jnp.bfloat16
    s1 = _sc_agg(y1.astype(bf16), src_r, dst_r, zero_rows).astype(f32)
    h1, y2 = _tc_layer(x_pad, s1, alpha, beta, gamma, W1a, W2a, bias_a,
                       last=False)
    s2 = _sc_agg(y2.astype(bf16), src_r, dst_r, zero_rows).astype(f32)
    h2, y3 = _tc_layer(h1, s2, alpha, beta, gamma, W1b, W2b, bias_b,
                       last=False)
    s3 = _sc_agg(y3.astype(bf16), src_r, dst_r, zero_rows).astype(f32)
    h3, _ = _tc_layer(h2, s3, alpha, beta, gamma, W1b, W2b, bias_b,
                      last=True)

    return jnp.concatenate(
        (h1[:N_NODES], h2[:N_NODES], h3[:N_NODES]), axis=-1)


# speedup vs baseline: 1.0431x; 1.0431x over previous
import dataclasses
import functools

import jax
import jax.numpy as jnp
from jax import lax
from jax.experimental import pallas as pl
from jax.experimental.pallas import tpu as pltpu
from jax.experimental.pallas import tpu_sc as plsc

N_NODES = 10000
N_PAD = 10000
N_EDGES = 320000
D = 128
NC = 2
NS = 16
NW = NC * NS
EPW = N_EDGES // NW
CHUNK = 125
NCHUNK = EPW // CHUNK
RPT = N_PAD // NS
LANES = 16

_mesh = plsc.VectorSubcoreMesh(core_axis_name="c", subcore_axis_name="s")

_sc_params = pltpu.CompilerParams(needs_layout_passes=False,
                                  use_tc_tiling_on_sc=False)


def _counts_body(src_hbm, dst_hbm, out_hbm, src_v, dst_v, cd_v, cs_v, cl_v, sem):
    cid = lax.axis_index("c")
    sid = lax.axis_index("s")
    wid = cid * NS + sid

    zeros16 = jnp.zeros((LANES,), jnp.float32)

    @pl.loop(0, N_PAD // LANES)
    def _zero(i):
        cd_v[0, pl.ds(i * LANES, LANES)] = zeros16
        cs_v[0, pl.ds(i * LANES, LANES)] = zeros16
        cl_v[0, pl.ds(i * LANES, LANES)] = zeros16

    cp1 = pltpu.async_copy(src_hbm.at[pl.ds(wid * EPW, EPW)], src_v, sem)
    cp2 = pltpu.async_copy(dst_hbm.at[pl.ds(wid * EPW, EPW)], dst_v, sem)
    cp1.wait()
    cp2.wait()

    ones16 = jnp.ones((LANES,), jnp.float32)
    zeros16i = jnp.zeros((LANES,), jnp.int32)

    @pl.loop(0, EPW // LANES)
    def _count(i):
        s16 = src_v[pl.ds(i * LANES, LANES)]
        d16 = dst_v[pl.ds(i * LANES, LANES)]
        neq = s16 != d16
        plsc.addupdate_scatter(cd_v, [zeros16i, d16], ones16, mask=neq)
        plsc.addupdate_scatter(cs_v, [zeros16i, s16], ones16, mask=neq)
        plsc.addupdate_scatter(cl_v, [zeros16i, d16], ones16,
                               mask=jnp.logical_not(neq))

    pltpu.sync_copy(cd_v, out_hbm.at[0, pl.ds(wid, 1)])
    pltpu.sync_copy(cs_v, out_hbm.at[1, pl.ds(wid, 1)])
    pltpu.sync_copy(cl_v, out_hbm.at[2, pl.ds(wid, 1)])


@jax.jit
def _sc_counts(src_flat, dst_flat):
    return pl.kernel(
        _counts_body,
        out_type=jax.ShapeDtypeStruct((3, NW, N_PAD), jnp.float32),
        mesh=_mesh,
        scratch_types=[
            pltpu.VMEM((EPW,), jnp.int32),
            pltpu.VMEM((EPW,), jnp.int32),
            pltpu.VMEM((1, N_PAD), jnp.float32),
            pltpu.VMEM((1, N_PAD), jnp.float32),
            pltpu.VMEM((1, N_PAD), jnp.float32),
            pltpu.SemaphoreType.DMA,
        ],
        compiler_params=_sc_params,
    )(src_flat, dst_flat)


NCHUNK = EPW // CHUNK

NBUF = 8
DEPTH = NBUF // 2


def _agg_body(y_hbm, src_hbm, dst_hbm, zero_hbm, out_hbm,
              src_v, dst_v, *rest):
    bufs = rest[:NBUF]
    acc_sh, semg, sems, semz = rest[NBUF:]
    cid = lax.axis_index("c")
    sid = lax.axis_index("s")
    wid = cid * NS + sid

    zcp = pltpu.async_copy(zero_hbm, acc_sh.at[pl.ds(sid * RPT, RPT)], semz)
    cp1 = pltpu.async_copy(src_hbm.at[wid], src_v, sem=semg)
    cp2 = pltpu.async_copy(dst_hbm.at[wid], dst_v, sem=semg)
    cp1.wait()
    cp2.wait()
    zcp.wait()
    plsc.subcore_barrier()

    def gather(j, b):
        pltpu.async_copy(y_hbm.at[src_v.at[j]], bufs[b], semg)

    def drain_gather(j, b):
        pltpu.make_async_copy(y_hbm.at[src_v.at[j]], bufs[b], semg).wait()

    def scatter(j, b):
        pltpu.async_copy(bufs[b], acc_sh.at[dst_v.at[j]], sems, add=True)

    def drain_scatter(j, b):
        pltpu.make_async_copy(bufs[b], acc_sh.at[dst_v.at[j]], sems).wait()

    for b in range(DEPTH):
        gather(b, b)

    @pl.loop(0, NCHUNK, step=NBUF)
    def _edges(base):
        for b in range(NBUF):
            j = base + b
            bn = (b + DEPTH) % NBUF

            @pl.when(j >= DEPTH)
            def _():
                drain_scatter(j - DEPTH, bn)

            @pl.when(j + DEPTH < NCHUNK)
            def _():
                gather(j + DEPTH, bn)

            drain_gather(j, b)
            scatter(j, b)

    for k in range(DEPTH):
        j = NCHUNK - DEPTH + k
        drain_scatter(j, j % NBUF)

    plsc.subcore_barrier()
    pltpu.sync_copy(acc_sh.at[pl.ds(sid * RPT, RPT)],
                    out_hbm.at[cid].at[pl.ds(sid * RPT, RPT)])


@jax.jit
def _sc_agg(y_bf16, src_r, dst_r, zero_rows):
    return pl.kernel(
        _agg_body,
        out_type=jax.ShapeDtypeStruct((NC, N_PAD, D), jnp.bfloat16),
        mesh=_mesh,
        scratch_types=[
            pltpu.VMEM((NCHUNK, CHUNK), jnp.int32),
            pltpu.VMEM((NCHUNK, CHUNK), jnp.int32),
        ] + [pltpu.VMEM((CHUNK, D), jnp.bfloat16)] * NBUF + [
            pltpu.VMEM_SHARED((N_PAD, D), jnp.bfloat16),
            pltpu.SemaphoreType.DMA,
            pltpu.SemaphoreType.DMA,
            pltpu.SemaphoreType.DMA,
        ],
        compiler_params=_sc_params,
    )(y_bf16, src_r, dst_r, zero_rows)


BP = 1000


def _prep_body(cnt_ref, x_ref, y_ref, a_ref, b_ref, g_ref):
    cnt = jnp.sum(cnt_ref[...], axis=1)
    deg = jnp.maximum(cnt[0], 1.0)
    deg_s = jnp.maximum(cnt[1], 1.0)
    cself = cnt[2]
    dinv = lax.rsqrt(deg)
    dinv_s = lax.rsqrt(deg_s)
    a_ref[...] = dinv[:, None]
    b_ref[...] = dinv_s[:, None]
    g_ref[...] = (cself * dinv * dinv_s)[:, None]
    y_ref[...] = (x_ref[...] * dinv_s[:, None]).astype(jnp.bfloat16)


@jax.jit
def _tc_prep(cnt_parts, x_pad):
    return pl.pallas_call(
        _prep_body,
        grid=(1,),
        in_specs=[
            pl.BlockSpec((3, NW, N_PAD), lambda i: (0, 0, 0)),
            pl.BlockSpec((N_PAD, D), lambda i: (0, 0)),
        ],
        out_specs=[
            pl.BlockSpec((N_PAD, D), lambda i: (0, 0)),
            pl.BlockSpec((N_PAD, 1), lambda i: (0, 0)),
            pl.BlockSpec((N_PAD, 1), lambda i: (0, 0)),
            pl.BlockSpec((N_PAD, 1), lambda i: (0, 0)),
        ],
        out_shape=[
            jax.ShapeDtypeStruct((N_PAD, D), jnp.bfloat16),
            jax.ShapeDtypeStruct((N_PAD, 1), jnp.float32),
            jax.ShapeDtypeStruct((N_PAD, 1), jnp.float32),
            jax.ShapeDtypeStruct((N_PAD, 1), jnp.float32),
        ],
    )(cnt_parts, x_pad)


def _layer_body(h_ref, s_ref, a_ref, b_ref, g_ref, w1_ref, w2_ref,
                bias_ref, o_ref, y_ref, *, last):
    h = h_ref[...]
    ssum = s_ref[0].astype(jnp.float32) + s_ref[1].astype(jnp.float32)
    agg = a_ref[...] * ssum - g_ref[...] * h
    u = h + agg
    v = h * agg
    o = (jnp.dot(u, w1_ref[...], preferred_element_type=jnp.float32)
         + jnp.dot(v, w2_ref[...], preferred_element_type=jnp.float32)
         + bias_ref[...])
    if not last:
        o = jnp.where(o > 0, o, 0.01 * o)
        y_ref[...] = (o * b_ref[...]).astype(jnp.bfloat16)
    o_ref[...] = o


@functools.partial(jax.jit, static_argnames=("last",))
def _tc_layer(h, s, alpha, beta, gamma, w1, w2, bias, *, last):
    grid = (N_PAD // BP,)
    out_shape = [jax.ShapeDtypeStruct((N_PAD, D), jnp.float32)]
    out_specs = [pl.BlockSpec((BP, D), lambda i: (i, 0))]
    if not last:
        out_shape.append(jax.ShapeDtypeStruct((N_PAD, D), jnp.bfloat16))
        out_specs.append(pl.BlockSpec((BP, D), lambda i: (i, 0)))
    else:
        out_shape.append(jax.ShapeDtypeStruct((8, 128), jnp.float32))
        out_specs.append(pl.BlockSpec((8, 128), lambda i: (0, 0)))
    return pl.pallas_call(
        functools.partial(_layer_body, last=last),
        grid=grid,
        in_specs=[
            pl.BlockSpec((BP, D), lambda i: (i, 0)),
            pl.BlockSpec((NC, BP, D), lambda i: (0, i, 0)),
            pl.BlockSpec((BP, 1), lambda i: (i, 0)),
            pl.BlockSpec((BP, 1), lambda i: (i, 0)),
            pl.BlockSpec((BP, 1), lambda i: (i, 0)),
            pl.BlockSpec((D, D), lambda i: (0, 0)),
            pl.BlockSpec((D, D), lambda i: (0, 0)),
            pl.BlockSpec((1, D), lambda i: (0, 0)),
        ],
        out_specs=out_specs,
        out_shape=out_shape,
    )(h, s, alpha, beta, gamma, w1, w2, bias)


def kernel(inputs, edge_index, W1a, b1a, W2a, b2a, W1b, b1b, W2b, b2b):
    src = edge_index[0].astype(jnp.int32)
    dst = edge_index[1].astype(jnp.int32)
    src_r = src.reshape(NW, NCHUNK, CHUNK)
    dst_r = dst.reshape(NW, NCHUNK, CHUNK)
    zero_rows = jnp.zeros((RPT, D), jnp.bfloat16)
    x_pad = inputs

    cnt_parts = _sc_counts(src, dst)
    y1, alpha, beta, gamma = _tc_prep(cnt_parts, x_pad)

    bias_a = (b1a + b2a).reshape(1, D)
    bias_b = (b1b + b2b).reshape(1, D)

    s1 = _sc_agg(y1, src_r, dst_r, zero_rows)
    h1, y2 = _tc_layer(x_pad, s1, alpha, beta, gamma, W1a, W2a, bias_a,
                       last=False)
    s2 = _sc_agg(y2, src_r, dst_r, zero_rows)
    h2, y3 = _tc_layer(h1, s2, alpha, beta, gamma, W1b, W2b, bias_b,
                       last=False)
    s3 = _sc_agg(y3, src_r, dst_r, zero_rows)
    h3, _ = _tc_layer(h2, s3, alpha, beta, gamma, W1b, W2b, bias_b,
                      last=True)

    return jnp.concatenate((h1, h2, h3), axis=-1)


# speedup vs baseline: 1.0646x; 1.0205x over previous
import dataclasses
import functools

import jax
import jax.numpy as jnp
from jax import lax
from jax.experimental import pallas as pl
from jax.experimental.pallas import tpu as pltpu
from jax.experimental.pallas import tpu_sc as plsc

N_NODES = 10000
N_PAD = 10000
N_EDGES = 320000
D = 128
NC = 2
NS = 16
NW = NC * NS
EPW = N_EDGES // NW
CHUNK = 125
NCHUNK = EPW // CHUNK
RPT = N_PAD // NS
LANES = 16

_mesh = plsc.VectorSubcoreMesh(core_axis_name="c", subcore_axis_name="s")

_sc_params = pltpu.CompilerParams(needs_layout_passes=False,
                                  use_tc_tiling_on_sc=False)
_sc_params_tc = pltpu.CompilerParams(needs_layout_passes=False,
                                     use_tc_tiling_on_sc=True)


def _counts_body(src_hbm, dst_hbm, out_hbm, src_v, dst_v, cd_v, cs_v, cl_v, sem):
    cid = lax.axis_index("c")
    sid = lax.axis_index("s")
    wid = cid * NS + sid

    zeros16 = jnp.zeros((LANES,), jnp.float32)

    @pl.loop(0, N_PAD // LANES)
    def _zero(i):
        cd_v[0, pl.ds(i * LANES, LANES)] = zeros16
        cs_v[0, pl.ds(i * LANES, LANES)] = zeros16
        cl_v[0, pl.ds(i * LANES, LANES)] = zeros16

    cp1 = pltpu.async_copy(src_hbm.at[pl.ds(wid * EPW, EPW)], src_v, sem)
    cp2 = pltpu.async_copy(dst_hbm.at[pl.ds(wid * EPW, EPW)], dst_v, sem)
    cp1.wait()
    cp2.wait()

    ones16 = jnp.ones((LANES,), jnp.float32)
    zeros16i = jnp.zeros((LANES,), jnp.int32)

    @pl.loop(0, EPW // LANES)
    def _count(i):
        s16 = src_v[pl.ds(i * LANES, LANES)]
        d16 = dst_v[pl.ds(i * LANES, LANES)]
        neq = s16 != d16
        plsc.addupdate_scatter(cd_v, [zeros16i, d16], ones16, mask=neq)
        plsc.addupdate_scatter(cs_v, [zeros16i, s16], ones16, mask=neq)
        plsc.addupdate_scatter(cl_v, [zeros16i, d16], ones16,
                               mask=jnp.logical_not(neq))

    pltpu.sync_copy(cd_v, out_hbm.at[0, pl.ds(wid, 1)])
    pltpu.sync_copy(cs_v, out_hbm.at[1, pl.ds(wid, 1)])
    pltpu.sync_copy(cl_v, out_hbm.at[2, pl.ds(wid, 1)])


@jax.jit
def _sc_counts(src_flat, dst_flat):
    return pl.kernel(
        _counts_body,
        out_type=jax.ShapeDtypeStruct((3, NW, N_PAD), jnp.float32),
        mesh=_mesh,
        scratch_types=[
            pltpu.VMEM((EPW,), jnp.int32),
            pltpu.VMEM((EPW,), jnp.int32),
            pltpu.VMEM((1, N_PAD), jnp.float32),
            pltpu.VMEM((1, N_PAD), jnp.float32),
            pltpu.VMEM((1, N_PAD), jnp.float32),
            pltpu.SemaphoreType.DMA,
        ],
        compiler_params=_sc_params_tc,
    )(src_flat, dst_flat)


NCHUNK = EPW // CHUNK

NBUF = 8
DEPTH = NBUF // 2


def _agg_body(y_hbm, src_hbm, dst_hbm, zero_hbm, out_hbm,
              src_v, dst_v, *rest):
    bufs = rest[:NBUF]
    acc_sh, semg, sems, semz = rest[NBUF:]
    cid = lax.axis_index("c")
    sid = lax.axis_index("s")
    wid = cid * NS + sid

    zcp = pltpu.async_copy(zero_hbm, acc_sh.at[pl.ds(sid * RPT, RPT)], semz)
    cp1 = pltpu.async_copy(src_hbm.at[wid], src_v, sem=semg)
    cp2 = pltpu.async_copy(dst_hbm.at[wid], dst_v, sem=semg)
    cp1.wait()
    cp2.wait()
    zcp.wait()
    plsc.subcore_barrier()

    def gather(j, b):
        pltpu.async_copy(y_hbm.at[src_v.at[j]], bufs[b], semg)

    def drain_gather(j, b):
        pltpu.make_async_copy(y_hbm.at[src_v.at[j]], bufs[b], semg).wait()

    def scatter(j, b):
        pltpu.async_copy(bufs[b], acc_sh.at[dst_v.at[j]], sems, add=True)

    def drain_scatter(j, b):
        pltpu.make_async_copy(bufs[b], acc_sh.at[dst_v.at[j]], sems).wait()

    for b in range(DEPTH):
        gather(b, b)

    @pl.loop(0, NCHUNK, step=NBUF)
    def _edges(base):
        for b in range(NBUF):
            j = base + b
            bn = (b + DEPTH) % NBUF

            @pl.when(j >= DEPTH)
            def _():
                drain_scatter(j - DEPTH, bn)

            @pl.when(j + DEPTH < NCHUNK)
            def _():
                gather(j + DEPTH, bn)

            drain_gather(j, b)
            scatter(j, b)

    for k in range(DEPTH):
        j = NCHUNK - DEPTH + k
        drain_scatter(j, j % NBUF)

    plsc.subcore_barrier()
    pltpu.sync_copy(acc_sh.at[pl.ds(sid * RPT, RPT)],
                    out_hbm.at[cid].at[pl.ds(sid * RPT, RPT)])


@jax.jit
def _sc_agg(y_bf16, src_r, dst_r, zero_rows):
    return pl.kernel(
        _agg_body,
        out_type=jax.ShapeDtypeStruct((NC, N_PAD, D), jnp.bfloat16),
        mesh=_mesh,
        scratch_types=[
            pltpu.VMEM((NCHUNK, CHUNK), jnp.int32),
            pltpu.VMEM((NCHUNK, CHUNK), jnp.int32),
        ] + [pltpu.VMEM((CHUNK, D), jnp.bfloat16)] * NBUF + [
            pltpu.VMEM_SHARED((N_PAD, D), jnp.bfloat16),
            pltpu.SemaphoreType.DMA,
            pltpu.SemaphoreType.DMA,
            pltpu.SemaphoreType.DMA,
        ],
        compiler_params=_sc_params,
    )(y_bf16, src_r, dst_r, zero_rows)


BP = 1000


def _prep_body(cnt_ref, x_ref, y_ref, a_ref, b_ref, g_ref):
    cnt = jnp.sum(cnt_ref[...], axis=1)
    deg = jnp.maximum(cnt[0], 1.0)
    deg_s = jnp.maximum(cnt[1], 1.0)
    cself = cnt[2]
    dinv = lax.rsqrt(deg)
    dinv_s = lax.rsqrt(deg_s)
    a_ref[...] = dinv[:, None]
    b_ref[...] = dinv_s[:, None]
    g_ref[...] = (cself * dinv * dinv_s)[:, None]
    y_ref[...] = (x_ref[...] * dinv_s[:, None]).astype(jnp.bfloat16)


@jax.jit
def _tc_prep(cnt_parts, x_pad):
    return pl.pallas_call(
        _prep_body,
        grid=(1,),
        in_specs=[
            pl.BlockSpec((3, NW, N_PAD), lambda i: (0, 0, 0)),
            pl.BlockSpec((N_PAD, D), lambda i: (0, 0)),
        ],
        out_specs=[
            pl.BlockSpec((N_PAD, D), lambda i: (0, 0)),
            pl.BlockSpec((N_PAD, 1), lambda i: (0, 0)),
            pl.BlockSpec((N_PAD, 1), lambda i: (0, 0)),
            pl.BlockSpec((N_PAD, 1), lambda i: (0, 0)),
        ],
        out_shape=[
            jax.ShapeDtypeStruct((N_PAD, D), jnp.bfloat16),
            jax.ShapeDtypeStruct((N_PAD, 1), jnp.float32),
            jax.ShapeDtypeStruct((N_PAD, 1), jnp.float32),
            jax.ShapeDtypeStruct((N_PAD, 1), jnp.float32),
        ],
    )(cnt_parts, x_pad)


def _layer_body(h_ref, s_ref, a_ref, b_ref, g_ref, w1_ref, w2_ref,
                bias_ref, o_ref, y_ref, *, last):
    h = h_ref[...]
    ssum = s_ref[0].astype(jnp.float32) + s_ref[1].astype(jnp.float32)
    agg = a_ref[...] * ssum - g_ref[...] * h
    u = h + agg
    v = h * agg
    o = (jnp.dot(u, w1_ref[...], preferred_element_type=jnp.float32)
         + jnp.dot(v, w2_ref[...], preferred_element_type=jnp.float32)
         + bias_ref[...])
    if not last:
        o = jnp.where(o > 0, o, 0.01 * o)
        y_ref[...] = (o * b_ref[...]).astype(jnp.bfloat16)
    o_ref[...] = o


@functools.partial(jax.jit, static_argnames=("last",))
def _tc_layer(h, s, alpha, beta, gamma, w1, w2, bias, *, last):
    grid = (N_PAD // BP,)
    out_shape = [jax.ShapeDtypeStruct((N_PAD, D), jnp.float32)]
    out_specs = [pl.BlockSpec((BP, D), lambda i: (i, 0))]
    if not last:
        out_shape.append(jax.ShapeDtypeStruct((N_PAD, D), jnp.bfloat16))
        out_specs.append(pl.BlockSpec((BP, D), lambda i: (i, 0)))
    else:
        out_shape.append(jax.ShapeDtypeStruct((8, 128), jnp.float32))
        out_specs.append(pl.BlockSpec((8, 128), lambda i: (0, 0)))
    return pl.pallas_call(
        functools.partial(_layer_body, last=last),
        grid=grid,
        in_specs=[
            pl.BlockSpec((BP, D), lambda i: (i, 0)),
            pl.BlockSpec((NC, BP, D), lambda i: (0, i, 0)),
            pl.BlockSpec((BP, 1), lambda i: (i, 0)),
            pl.BlockSpec((BP, 1), lambda i: (i, 0)),
            pl.BlockSpec((BP, 1), lambda i: (i, 0)),
            pl.BlockSpec((D, D), lambda i: (0, 0)),
            pl.BlockSpec((D, D), lambda i: (0, 0)),
            pl.BlockSpec((1, D), lambda i: (0, 0)),
        ],
        out_specs=out_specs,
        out_shape=out_shape,
    )(h, s, alpha, beta, gamma, w1, w2, bias)


def kernel(inputs, edge_index, W1a, b1a, W2a, b2a, W1b, b1b, W2b, b2b):
    src = edge_index[0].astype(jnp.int32)
    dst = edge_index[1].astype(jnp.int32)
    src_r = src.reshape(NW, NCHUNK, CHUNK)
    dst_r = dst.reshape(NW, NCHUNK, CHUNK)
    zero_rows = jnp.zeros((RPT, D), jnp.bfloat16)
    x_pad = inputs

    cnt_parts = _sc_counts(src, dst)
    y1, alpha, beta, gamma = _tc_prep(cnt_parts, x_pad)

    bias_a = (b1a + b2a).reshape(1, D)
    bias_b = (b1b + b2b).reshape(1, D)

    s1 = _sc_agg(y1, src_r, dst_r, zero_rows)
    h1, y2 = _tc_layer(x_pad, s1, alpha, beta, gamma, W1a, W2a, bias_a,
                       last=False)
    s2 = _sc_agg(y2, src_r, dst_r, zero_rows)
    h2, y3 = _tc_layer(h1, s2, alpha, beta, gamma, W1b, W2b, bias_b,
                       last=False)
    s3 = _sc_agg(y3, src_r, dst_r, zero_rows)
    h3, _ = _tc_layer(h2, s3, alpha, beta, gamma, W1b, W2b, bias_b,
                      last=True)

    return jnp.concatenate((h1, h2, h3), axis=-1)


# speedup vs baseline: 1.0816x; 1.0160x over previous
import dataclasses
import functools

import jax
import jax.numpy as jnp
from jax import lax
from jax.experimental import pallas as pl
from jax.experimental.pallas import tpu as pltpu
from jax.experimental.pallas import tpu_sc as plsc

N_NODES = 10000
N_PAD = 10000
N_EDGES = 320000
D = 128
NC = 2
NS = 16
NW = NC * NS
EPW = N_EDGES // NW
CHUNK = 125
NCHUNK = EPW // CHUNK
RPT = N_PAD // NS
LANES = 16

_mesh = plsc.VectorSubcoreMesh(core_axis_name="c", subcore_axis_name="s")

_sc_params = pltpu.CompilerParams(needs_layout_passes=False,
                                  use_tc_tiling_on_sc=False)
_sc_params_tc = pltpu.CompilerParams(needs_layout_passes=False,
                                     use_tc_tiling_on_sc=True)


def _counts_body(src_hbm, dst_hbm, out_hbm, src_v, dst_v, cd_v, cs_v, cl_v, sem):
    cid = lax.axis_index("c")
    sid = lax.axis_index("s")
    wid = cid * NS + sid

    zeros16 = jnp.zeros((LANES,), jnp.float32)

    @pl.loop(0, N_PAD // LANES)
    def _zero(i):
        cd_v[0, pl.ds(i * LANES, LANES)] = zeros16
        cs_v[0, pl.ds(i * LANES, LANES)] = zeros16
        cl_v[0, pl.ds(i * LANES, LANES)] = zeros16

    cp1 = pltpu.async_copy(src_hbm.at[pl.ds(wid * EPW, EPW)], src_v, sem)
    cp2 = pltpu.async_copy(dst_hbm.at[pl.ds(wid * EPW, EPW)], dst_v, sem)
    cp1.wait()
    cp2.wait()

    ones16 = jnp.ones((LANES,), jnp.float32)
    zeros16i = jnp.zeros((LANES,), jnp.int32)

    @pl.loop(0, EPW // LANES)
    def _count(i):
        s16 = src_v[pl.ds(i * LANES, LANES)]
        d16 = dst_v[pl.ds(i * LANES, LANES)]
        neq = s16 != d16
        plsc.addupdate_scatter(cd_v, [zeros16i, d16], ones16, mask=neq)
        plsc.addupdate_scatter(cs_v, [zeros16i, s16], ones16, mask=neq)
        plsc.addupdate_scatter(cl_v, [zeros16i, d16], ones16,
                               mask=jnp.logical_not(neq))

    pltpu.sync_copy(cd_v, out_hbm.at[0, pl.ds(wid, 1)])
    pltpu.sync_copy(cs_v, out_hbm.at[1, pl.ds(wid, 1)])
    pltpu.sync_copy(cl_v, out_hbm.at[2, pl.ds(wid, 1)])


@jax.jit
def _sc_counts(src_flat, dst_flat):
    return pl.kernel(
        _counts_body,
        out_type=jax.ShapeDtypeStruct((3, NW, N_PAD), jnp.float32),
        mesh=_mesh,
        scratch_types=[
            pltpu.VMEM((EPW,), jnp.int32),
            pltpu.VMEM((EPW,), jnp.int32),
            pltpu.VMEM((1, N_PAD), jnp.float32),
            pltpu.VMEM((1, N_PAD), jnp.float32),
            pltpu.VMEM((1, N_PAD), jnp.float32),
            pltpu.SemaphoreType.DMA,
        ],
        compiler_params=_sc_params_tc,
    )(src_flat, dst_flat)


NCHUNK = EPW // CHUNK

NBUF = 8
DEPTH = NBUF // 2


def _agg_body(y_hbm, src_hbm, dst_hbm, zero_hbm, out_hbm,
              src_v, dst_v, *rest):
    bufs = rest[:NBUF]
    acc_sh, semg, sems, semz = rest[NBUF:]
    cid = lax.axis_index("c")
    sid = lax.axis_index("s")
    wid = cid * NS + sid

    zcp = pltpu.async_copy(zero_hbm, acc_sh.at[pl.ds(sid * RPT, RPT)], semz)
    cp1 = pltpu.async_copy(src_hbm.at[wid], src_v, sem=semg)
    cp2 = pltpu.async_copy(dst_hbm.at[wid], dst_v, sem=semg)
    cp1.wait()
    cp2.wait()
    zcp.wait()
    plsc.subcore_barrier()

    def gather(j, b):
        pltpu.async_copy(y_hbm.at[src_v.at[j]], bufs[b], semg)

    def drain_gather(j, b):
        pltpu.make_async_copy(y_hbm.at[src_v.at[j]], bufs[b], semg).wait()

    def scatter(j, b):
        pltpu.async_copy(bufs[b], acc_sh.at[dst_v.at[j]], sems, add=True)

    def drain_scatter(j, b):
        pltpu.make_async_copy(bufs[b], acc_sh.at[dst_v.at[j]], sems).wait()

    for b in range(DEPTH):
        gather(b, b)

    @pl.loop(0, NCHUNK, step=NBUF)
    def _edges(base):
        for b in range(NBUF):
            j = base + b
            bn = (b + DEPTH) % NBUF

            @pl.when(j >= DEPTH)
            def _():
                drain_scatter(j - DEPTH, bn)

            @pl.when(j + DEPTH < NCHUNK)
            def _():
                gather(j + DEPTH, bn)

            drain_gather(j, b)
            scatter(j, b)

    for k in range(DEPTH):
        j = NCHUNK - DEPTH + k
        drain_scatter(j, j % NBUF)

    plsc.subcore_barrier()
    pltpu.sync_copy(acc_sh.at[pl.ds(sid * RPT, RPT)],
                    out_hbm.at[cid].at[pl.ds(sid * RPT, RPT)])


@jax.jit
def _sc_agg(y_bf16, src_r, dst_r, zero_rows):
    return pl.kernel(
        _agg_body,
        out_type=jax.ShapeDtypeStruct((NC, N_PAD, D), jnp.bfloat16),
        mesh=_mesh,
        scratch_types=[
            pltpu.VMEM((NCHUNK, CHUNK), jnp.int32),
            pltpu.VMEM((NCHUNK, CHUNK), jnp.int32),
        ] + [pltpu.VMEM((CHUNK, D), jnp.bfloat16)] * NBUF + [
            pltpu.VMEM_SHARED((N_PAD, D), jnp.bfloat16),
            pltpu.SemaphoreType.DMA,
            pltpu.SemaphoreType.DMA,
            pltpu.SemaphoreType.DMA,
        ],
        compiler_params=_sc_params,
    )(y_bf16, src_r, dst_r, zero_rows)


BP = 2000


def _prep_body(cnt_ref, x_ref, y_ref, a_ref, b_ref, g_ref):
    cnt = jnp.sum(cnt_ref[...], axis=1)
    deg = jnp.maximum(cnt[0], 1.0)
    deg_s = jnp.maximum(cnt[1], 1.0)
    cself = cnt[2]
    dinv = lax.rsqrt(deg)
    dinv_s = lax.rsqrt(deg_s)
    a_ref[...] = dinv[:, None]
    b_ref[...] = dinv_s[:, None]
    g_ref[...] = (cself * dinv * dinv_s)[:, None]
    y_ref[...] = (x_ref[...] * dinv_s[:, None]).astype(jnp.bfloat16)


@jax.jit
def _tc_prep(cnt_parts, x_pad):
    return pl.pallas_call(
        _prep_body,
        grid=(1,),
        in_specs=[
            pl.BlockSpec((3, NW, N_PAD), lambda i: (0, 0, 0)),
            pl.BlockSpec((N_PAD, D), lambda i: (0, 0)),
        ],
        out_specs=[
            pl.BlockSpec((N_PAD, D), lambda i: (0, 0)),
            pl.BlockSpec((N_PAD, 1), lambda i: (0, 0)),
            pl.BlockSpec((N_PAD, 1), lambda i: (0, 0)),
            pl.BlockSpec((N_PAD, 1), lambda i: (0, 0)),
        ],
        out_shape=[
            jax.ShapeDtypeStruct((N_PAD, D), jnp.bfloat16),
            jax.ShapeDtypeStruct((N_PAD, 1), jnp.float32),
            jax.ShapeDtypeStruct((N_PAD, 1), jnp.float32),
            jax.ShapeDtypeStruct((N_PAD, 1), jnp.float32),
        ],
    )(cnt_parts, x_pad)


def _layer_body(h_ref, s_ref, a_ref, b_ref, g_ref, w1_ref, w2_ref,
                bias_ref, o_ref, y_ref, *, last):
    h = h_ref[...]
    ssum = s_ref[0].astype(jnp.float32) + s_ref[1].astype(jnp.float32)
    agg = a_ref[...] * ssum - g_ref[...] * h
    u = h + agg
    v = h * agg
    bf = jnp.bfloat16
    o = (jnp.dot(u.astype(bf), w1_ref[...].astype(bf),
                 preferred_element_type=jnp.float32)
         + jnp.dot(v.astype(bf), w2_ref[...].astype(bf),
                   preferred_element_type=jnp.float32)
         + bias_ref[...])
    if not last:
        o = jnp.where(o > 0, o, 0.01 * o)
        y_ref[...] = (o * b_ref[...]).astype(jnp.bfloat16)
    o_ref[...] = o


@functools.partial(jax.jit, static_argnames=("last",))
def _tc_layer(h, s, alpha, beta, gamma, w1, w2, bias, *, last):
    grid = (N_PAD // BP,)
    out_shape = [jax.ShapeDtypeStruct((N_PAD, D), jnp.float32)]
    out_specs = [pl.BlockSpec((BP, D), lambda i: (i, 0))]
    if not last:
        out_shape.append(jax.ShapeDtypeStruct((N_PAD, D), jnp.bfloat16))
        out_specs.append(pl.BlockSpec((BP, D), lambda i: (i, 0)))
    else:
        out_shape.append(jax.ShapeDtypeStruct((8, 128), jnp.float32))
        out_specs.append(pl.BlockSpec((8, 128), lambda i: (0, 0)))
    return pl.pallas_call(
        functools.partial(_layer_body, last=last),
        grid=grid,
        in_specs=[
            pl.BlockSpec((BP, D), lambda i: (i, 0)),
            pl.BlockSpec((NC, BP, D), lambda i: (0, i, 0)),
            pl.BlockSpec((BP, 1), lambda i: (i, 0)),
            pl.BlockSpec((BP, 1), lambda i: (i, 0)),
            pl.BlockSpec((BP, 1), lambda i: (i, 0)),
            pl.BlockSpec((D, D), lambda i: (0, 0)),
            pl.BlockSpec((D, D), lambda i: (0, 0)),
            pl.BlockSpec((1, D), lambda i: (0, 0)),
        ],
        out_specs=out_specs,
        out_shape=out_shape,
    )(h, s, alpha, beta, gamma, w1, w2, bias)


def kernel(inputs, edge_index, W1a, b1a, W2a, b2a, W1b, b1b, W2b, b2b):
    src = edge_index[0].astype(jnp.int32)
    dst = edge_index[1].astype(jnp.int32)
    src_r = src.reshape(NW, NCHUNK, CHUNK)
    dst_r = dst.reshape(NW, NCHUNK, CHUNK)
    zero_rows = jnp.zeros((RPT, D), jnp.bfloat16)
    x_pad = inputs

    cnt_parts = _sc_counts(src, dst)
    y1, alpha, beta, gamma = _tc_prep(cnt_parts, x_pad)

    bias_a = (b1a + b2a).reshape(1, D)
    bias_b = (b1b + b2b).reshape(1, D)

    s1 = _sc_agg(y1, src_r, dst_r, zero_rows)
    h1, y2 = _tc_layer(x_pad, s1, alpha, beta, gamma, W1a, W2a, bias_a,
                       last=False)
    s2 = _sc_agg(y2, src_r, dst_r, zero_rows)
    h2, y3 = _tc_layer(h1, s2, alpha, beta, gamma, W1b, W2b, bias_b,
                       last=False)
    s3 = _sc_agg(y3, src_r, dst_r, zero_rows)
    h3, _ = _tc_layer(h2, s3, alpha, beta, gamma, W1b, W2b, bias_b,
                      last=True)

    return jnp.concatenate((h1, h2, h3), axis=-1)
